# baseline (speedup 1.0000x reference)
import functools

import jax
import jax.numpy as jnp
from jax import lax
from jax.experimental import pallas as pl
from jax.experimental.pallas import tpu as pltpu
from jax.experimental.pallas import tpu_sc as plsc

BATCH = 16384
EMB = 64
NROWS = 100000

_NC = 2
_NS = 16
_NW = _NC * _NS
_BPW = BATCH // _NW
_CH = 128
_NCHUNK = _BPW // _CH
_SUB = 2 * _CH

_LB = 24576


def _prep_body(gt, mt, out):
    x = jnp.concatenate([gt[...], mt[...]], axis=0)
    r = lax.broadcasted_iota(jnp.int32, (2 * EMB, 2 * EMB), 0)
    c = lax.broadcasted_iota(jnp.int32, (2 * EMB, 2 * EMB), 1)
    ident = jnp.where(r == c, 1.0, 0.0).astype(jnp.float32)
    out[...] = lax.dot_general(x, ident, (((0,), (0,)), ((), ())),
                               preferred_element_type=jnp.float32)


def _tc_prep(gt, mt):
    grid = (pl.cdiv(NROWS, _LB),)
    return pl.pallas_call(
        _prep_body,
        grid=grid,
        in_specs=[
            pl.BlockSpec((EMB, _LB), lambda i: (0, i)),
            pl.BlockSpec((EMB, _LB), lambda i: (0, i)),
        ],
        out_specs=pl.BlockSpec((_LB, 2 * EMB), lambda i: (i, 0)),
        out_shape=jax.ShapeDtypeStruct((NROWS, 2 * EMB), jnp.float32),
        compiler_params=pltpu.CompilerParams(
            dimension_semantics=("arbitrary",),
            vmem_limit_bytes=60 * 1024 * 1024),
    )(gt, mt)


def _sc_gather(ids3, tab):
    mesh = plsc.VectorSubcoreMesh(core_axis_name="c", subcore_axis_name="s")
    nchunk = ids3.shape[1]
    bpw = nchunk * _CH
    nsub = nchunk // 2
    batch = _NW * bpw

    @functools.partial(
        pl.kernel,
        out_type=jax.ShapeDtypeStruct((batch, 2 * EMB), jnp.float32),
        mesh=mesh,
        scratch_types=[
            pltpu.VMEM((nchunk, _CH), jnp.int32),
            pltpu.VMEM((_SUB, 2 * EMB), jnp.float32),
            pltpu.VMEM((_SUB, 2 * EMB), jnp.float32),
            pltpu.SemaphoreType.DMA,
            pltpu.SemaphoreType.DMA,
        ],
    )
    def sc_k(ids_h, tab_h, out, iv, rows0, rows1, s0, s1):
        wid = lax.axis_index("s") * _NC + lax.axis_index("c")
        base = wid * bpw
        pltpu.sync_copy(ids_h.at[wid], iv)
        bufs = (rows0, rows1)
        sems = (s0, s1)

        def fire(h):
            buf, sem = bufs[h % 2], sems[h % 2]
            cs = []
            for j in range(2):
                cs.append(pltpu.async_copy(
                    tab_h.at[iv.at[2 * h + j]],
                    buf.at[pl.ds(j * _CH, _CH)], sem))
            return cs

        pend = [fire(0)]
        if nsub > 1:
            pend.append(fire(1))
        for h in range(nsub):
            for c in pend[h % 2]:
                c.wait()
            pltpu.sync_copy(bufs[h % 2],
                            out.at[pl.ds(base + h * _SUB, _SUB)])
            if h + 2 < nsub:
                pend[h % 2] = fire(h + 2)

    return sc_k(ids3, tab)


_BB = 2048


def _tc_body(ru, ri, w1u, w1i, b1, w2, b2, w3, b3, wog, woh, bo, out):
    u = ru[...]
    v = ri[...]
    h = jnp.dot(u[:, EMB:], w1u[...], preferred_element_type=jnp.float32)
    h = h + jnp.dot(v[:, EMB:], w1i[...], preferred_element_type=jnp.float32)
    h = jnp.maximum(h + b1[...], 0.0)
    h = jnp.maximum(
        jnp.dot(h, w2[...], preferred_element_type=jnp.float32) + b2[...], 0.0)
    h = jnp.maximum(
        jnp.dot(h, w3[...], preferred_element_type=jnp.float32) + b3[...], 0.0)
    g = u[:, :EMB] * v[:, :EMB]
    p = jnp.dot(g, wog[...], preferred_element_type=jnp.float32)
    p = p + jnp.dot(h, woh[...], preferred_element_type=jnp.float32)
    out[...] = p + bo[...]


def _tc_mlp(ru, ri, w1u, w1i, b1, w2, b2, w3, b3, wog, woh, bo):
    grid = (ru.shape[0] // _BB,)
    fixed = lambda i: (0, 0)
    row = lambda i: (i, 0)
    in_specs = [
        pl.BlockSpec((_BB, 2 * EMB), row),
        pl.BlockSpec((_BB, 2 * EMB), row),
        pl.BlockSpec(w1u.shape, fixed),
        pl.BlockSpec(w1i.shape, fixed),
        pl.BlockSpec(b1.shape, fixed),
        pl.BlockSpec(w2.shape, fixed),
        pl.BlockSpec(b2.shape, fixed),
        pl.BlockSpec(w3.shape, fixed),
        pl.BlockSpec(b3.shape, fixed),
        pl.BlockSpec(wog.shape, fixed),
        pl.BlockSpec(woh.shape, fixed),
        pl.BlockSpec(bo.shape, fixed),
    ]
    return pl.pallas_call(
        _tc_body,
        grid=grid,
        in_specs=in_specs,
        out_specs=pl.BlockSpec((_BB, 1), row),
        out_shape=jax.ShapeDtypeStruct((ru.shape[0], 1), jnp.float32),
        compiler_params=pltpu.CompilerParams(
            dimension_semantics=("parallel",)),
    )(ru, ri, w1u, w1i, b1, w2, b2, w3, b3, wog, woh, bo)


def kernel(user_ids, item_ids, gmf_user, gmf_item, mlp_user, mlp_item,
           W1, b1, W2, b2, W3, b3, Wo, bo):
    half = BATCH // 2
    uid3 = user_ids.astype(jnp.int32).reshape(_NW, _NCHUNK, _CH)
    iid = item_ids.astype(jnp.int32)
    iid3a = iid[:half].reshape(_NW, _NCHUNK // 2, _CH)
    iid3b = iid[half:].reshape(_NW, _NCHUNK // 2, _CH)
    utab = _tc_prep(gmf_user.T, mlp_user.T)
    ru = _sc_gather(uid3, utab)
    itab = _tc_prep(gmf_item.T, mlp_item.T)
    ri_a = _sc_gather(iid3a, itab)
    ri_b = _sc_gather(iid3b, itab)
    weights = (W1[:EMB], W1[EMB:], b1.reshape(1, -1),
               W2, b2.reshape(1, -1),
               W3, b3.reshape(1, -1),
               Wo[:EMB], Wo[EMB:], bo.reshape(1, 1))
    pred_a = _tc_mlp(ru[:half], ri_a, *weights)
    pred_b = _tc_mlp(ru[half:], ri_b, *weights)
    return jnp.concatenate([pred_a[:, 0], pred_b[:, 0]])


# speedup vs baseline: 1.1349x; 1.1349x over previous
import functools

import jax
import jax.numpy as jnp
from jax import lax
from jax.experimental import pallas as pl
from jax.experimental.pallas import tpu as pltpu
from jax.experimental.pallas import tpu_sc as plsc

BATCH = 16384
EMB = 64
NROWS = 100000

_NC = 2
_NS = 16
_NW = _NC * _NS
_BPW = BATCH // _NW
_CH = 128
_NCHUNK = _BPW // _CH
_SUB = 2 * _CH

_LB = 24576


def _prep_body(gt, mt, out):
    x = jnp.concatenate([gt[...], mt[...]], axis=0)
    r = lax.broadcasted_iota(jnp.int32, (2 * EMB, 2 * EMB), 0)
    c = lax.broadcasted_iota(jnp.int32, (2 * EMB, 2 * EMB), 1)
    ident = jnp.where(r == c, 1.0, 0.0).astype(jnp.float32)
    out[...] = lax.dot_general(x, ident, (((0,), (0,)), ((), ())),
                               preferred_element_type=jnp.float32)


def _tc_prep(gt, mt):
    grid = (pl.cdiv(NROWS, _LB),)
    return pl.pallas_call(
        _prep_body,
        grid=grid,
        in_specs=[
            pl.BlockSpec((EMB, _LB), lambda i: (0, i)),
            pl.BlockSpec((EMB, _LB), lambda i: (0, i)),
        ],
        out_specs=pl.BlockSpec((_LB, 2 * EMB), lambda i: (i, 0)),
        out_shape=jax.ShapeDtypeStruct((NROWS, 2 * EMB), jnp.float32),
        compiler_params=pltpu.CompilerParams(
            dimension_semantics=("arbitrary",),
            vmem_limit_bytes=60 * 1024 * 1024),
    )(gt, mt)


def _sc_gather(ids3, tab):
    mesh = plsc.VectorSubcoreMesh(core_axis_name="c", subcore_axis_name="s")

    @functools.partial(
        pl.kernel,
        out_type=jax.ShapeDtypeStruct((BATCH, 2 * EMB), jnp.float32),
        mesh=mesh,
        scratch_types=[
            pltpu.VMEM((_NCHUNK, _CH), jnp.int32),
            pltpu.VMEM((_SUB, 2 * EMB), jnp.float32),
            pltpu.VMEM((_SUB, 2 * EMB), jnp.float32),
            pltpu.SemaphoreType.DMA,
            pltpu.SemaphoreType.DMA,
        ],
    )
    def sc_k(ids_h, tab_h, out, iv, rows0, rows1, s0, s1):
        wid = lax.axis_index("s") * _NC + lax.axis_index("c")
        base = wid * _BPW
        pltpu.sync_copy(ids_h.at[wid], iv)
        bufs = (rows0, rows1)
        sems = (s0, s1)

        def fire(h):
            buf, sem = bufs[h % 2], sems[h % 2]
            cs = []
            for j in range(2):
                cs.append(pltpu.async_copy(
                    tab_h.at[iv.at[2 * h + j]],
                    buf.at[pl.ds(j * _CH, _CH)], sem))
            return cs

        pend = [fire(0), fire(1)]
        for h in range(2):
            for c in pend[h]:
                c.wait()
            pltpu.sync_copy(bufs[h],
                            out.at[pl.ds(base + h * _SUB, _SUB)])

    return sc_k(ids3, tab)


_BB = 4096


def _tc_body(ru, ri, w1u, w1i, b1, w2, b2, w3, b3, wog, woh, bo, out):
    u = ru[...]
    v = ri[...]
    h = jnp.dot(u[:, EMB:], w1u[...], preferred_element_type=jnp.float32)
    h = h + jnp.dot(v[:, EMB:], w1i[...], preferred_element_type=jnp.float32)
    h = jnp.maximum(h + b1[...], 0.0)
    h = jnp.maximum(
        jnp.dot(h, w2[...], preferred_element_type=jnp.float32) + b2[...], 0.0)
    h = jnp.maximum(
        jnp.dot(h, w3[...], preferred_element_type=jnp.float32) + b3[...], 0.0)
    g = u[:, :EMB] * v[:, :EMB]
    p = jnp.dot(g, wog[...], preferred_element_type=jnp.float32)
    p = p + jnp.dot(h, woh[...], preferred_element_type=jnp.float32)
    out[...] = p + bo[...]


def _tc_mlp(ru, ri, w1u, w1i, b1, w2, b2, w3, b3, wog, woh, bo):
    grid = (BATCH // _BB,)
    fixed = lambda i: (0, 0)
    row = lambda i: (i, 0)
    in_specs = [
        pl.BlockSpec((_BB, 2 * EMB), row),
        pl.BlockSpec((_BB, 2 * EMB), row),
        pl.BlockSpec(w1u.shape, fixed),
        pl.BlockSpec(w1i.shape, fixed),
        pl.BlockSpec(b1.shape, fixed),
        pl.BlockSpec(w2.shape, fixed),
        pl.BlockSpec(b2.shape, fixed),
        pl.BlockSpec(w3.shape, fixed),
        pl.BlockSpec(b3.shape, fixed),
        pl.BlockSpec(wog.shape, fixed),
        pl.BlockSpec(woh.shape, fixed),
        pl.BlockSpec(bo.shape, fixed),
    ]
    return pl.pallas_call(
        _tc_body,
        grid=grid,
        in_specs=in_specs,
        out_specs=pl.BlockSpec((_BB, 1), row),
        out_shape=jax.ShapeDtypeStruct((BATCH, 1), jnp.float32),
        compiler_params=pltpu.CompilerParams(
            dimension_semantics=("parallel",)),
    )(ru, ri, w1u, w1i, b1, w2, b2, w3, b3, wog, woh, bo)


def kernel(user_ids, item_ids, gmf_user, gmf_item, mlp_user, mlp_item,
           W1, b1, W2, b2, W3, b3, Wo, bo):
    uid3 = user_ids.astype(jnp.int32).reshape(_NW, _NCHUNK, _CH)
    iid3 = item_ids.astype(jnp.int32).reshape(_NW, _NCHUNK, _CH)
    utab = _tc_prep(gmf_user.T, mlp_user.T)
    ru = _sc_gather(uid3, utab)
    itab = _tc_prep(gmf_item.T, mlp_item.T)
    ri = _sc_gather(iid3, itab)
    pred = _tc_mlp(
        ru, ri,
        W1[:EMB], W1[EMB:], b1.reshape(1, -1),
        W2, b2.reshape(1, -1),
        W3, b3.reshape(1, -1),
        Wo[:EMB], Wo[EMB:], bo.reshape(1, 1),
    )
    return pred[:, 0]
